# baseline (speedup 1.0000x reference)
import jax
import jax.numpy as jnp
from jax import lax
from jax.experimental import pallas as pl
from jax.experimental.pallas import tpu as pltpu
from jax.experimental.pallas import tpu_sc as plsc

N = 10000
E = 320000
F = 128
NS = 16
L = 16
EB = 128
NB = 158
E_PAD = NB * EB * NS
N_PAD = 10240
RPT = N_PAD // NS
BLK = 512


def _d_body(x_ref, dum_ref, o_ref):
    o_ref[...] = jnp.dot(x_ref[...], dum_ref[...],
                         preferred_element_type=jnp.float32)


def _final_body(x_ref, b_acc_ref, s_acc_ref, cnt_ref, wd_ref, w1_ref, ws_ref,
                bias_ref, o_ref):
    acc = jnp.dot(b_acc_ref[...], wd_ref[...],
                  preferred_element_type=jnp.float32)
    acc = acc + jnp.dot(s_acc_ref[...], w1_ref[...],
                        preferred_element_type=jnp.float32)
    acc = acc + jnp.dot(x_ref[...], ws_ref[...],
                        preferred_element_type=jnp.float32)
    cnt = jnp.sum(cnt_ref[...], axis=0)[:, None]
    conv = acc / (cnt + 1.0) + bias_ref[...]
    o_ref[...] = x_ref[...] + jnp.maximum(conv, 0.0)


def _sc_edges(src_ref, dst_ref, x_ref, d_ref, c01_ref,
              b_out, s_out, cnt_out,
              acc, rows0, rows1, rout, srcb0, srcb1, dstb0, dstb1, wb, dc,
              c01v, gsem0, gsem1):
    cid = lax.axis_index("c")
    wid = lax.axis_index("s")
    rows = (rows0, rows1)
    srcb = (srcb0, srcb1)
    dstb = (dstb0, dstb1)
    gsem = (gsem0, gsem1)
    dloc = dc
    cntloc = dc

    def _zrow(r, _):
        for t in range(F // L):
            rout[r, pl.ds(t * L, L)] = jnp.zeros((L,), jnp.float32)
        return 0
    lax.fori_loop(0, EB, _zrow, 0)
    for k in range(RPT // EB):
        pltpu.sync_copy(rout, acc.at[pl.ds(wid * RPT + k * EB, EB)])

    @pl.when(cid == 0)
    def _():
        pltpu.sync_copy(d_ref, dloc.at[pl.ds(0, N)])

    @pl.when(cid == 1)
    def _():
        def _zcnt(r, _):
            cntloc[pl.ds(r * L, L)] = jnp.zeros((L,), jnp.float32)
            return 0
        lax.fori_loop(0, N_PAD // L, _zcnt, 0)

    pltpu.sync_copy(c01_ref, c01v)
    plsc.subcore_barrier()

    base = wid * (NB * EB)

    def _fetch(i, b):
        off = base + i * EB
        pltpu.sync_copy(src_ref.at[pl.ds(off, EB)], srcb[b])
        pltpu.sync_copy(dst_ref.at[pl.ds(off, EB)], dstb[b])
        pltpu.async_copy(x_ref.at[srcb[b]], rows[b], gsem[b])

    def _gwait(b):
        pltpu.make_async_copy(x_ref.at[srcb[b]], rows[b], gsem[b]).wait()

    def _compute_sc0(b):
        c01 = c01v[...]
        for g in range(EB // L):
            sv = srcb[b][pl.ds(g * L, L)]
            dv = dstb[b][pl.ds(g * L, L)]
            dsv = plsc.load_gather(dloc, [sv])
            ddv = plsc.load_gather(dloc, [dv])
            z = dsv - ddv + c01
            w = 1.0 / (1.0 + jnp.exp(-z))
            w = jnp.where(sv == dv, 0.0, w)
            wb[pl.ds(g * L, L)] = w

        def _scale(g, _):
            wv = wb[pl.ds(g * L, L)]
            for j in range(L):
                wj = wv[j]
                row = g * L + j
                for t in range(F // (2 * L)):
                    v32 = rows[b][row, pl.ds(t * L, L)]
                    vb = plsc.bitcast(v32, jnp.bfloat16)
                    va, vc = plsc.unpack(
                        vb, format=plsc.PackFormat.INTERLEAVED,
                        preferred_element_type=jnp.float32)
                    rout[row, pl.ds(t * 2 * L, L)] = va * wj
                    rout[row, pl.ds(t * 2 * L + L, L)] = vc * wj
            return 0
        lax.fori_loop(0, EB // L, _scale, 0)

    def _compute_sc1(b):
        ones = jnp.ones((L,), jnp.float32)
        for g in range(EB // L):
            sl = pl.ds(g * L, L)
            sv = srcb[b][sl]
            dv = dstb[b][sl]
            valid = sv != dv
            dstb[b][sl] = jnp.where(valid, dv, N)
            plsc.addupdate_scatter(cntloc, [dv], ones, mask=valid)

        def _unpk(g, _):
            for j in range(L):
                row = g * L + j
                for t in range(F // (2 * L)):
                    v32 = rows[b][row, pl.ds(t * L, L)]
                    vb = plsc.bitcast(v32, jnp.bfloat16)
                    va, vc = plsc.unpack(
                        vb, format=plsc.PackFormat.INTERLEAVED,
                        preferred_element_type=jnp.float32)
                    rout[row, pl.ds(t * 2 * L, L)] = va
                    rout[row, pl.ds(t * 2 * L + L, L)] = vc
            return 0
        lax.fori_loop(0, EB // L, _unpk, 0)

    def _make_loop(compute):
        def _body(g, _):
            for b in (0, 1):
                i = 2 * g + b
                _gwait(b)

                @pl.when(i + 1 < NB)
                def _():
                    _fetch(i + 1, 1 - b)
                compute(b)
                pltpu.sync_copy(rout, acc.at[dstb[b]], add=True)
            return 0
        return _body

    _fetch(0, 0)

    @pl.when(cid == 0)
    def _():
        lax.fori_loop(0, NB // 2, _make_loop(_compute_sc0), 0)

    @pl.when(cid == 1)
    def _():
        lax.fori_loop(0, NB // 2, _make_loop(_compute_sc1), 0)

    plsc.subcore_barrier()
    stripe = pl.ds(wid * RPT, RPT)

    @pl.when(cid == 0)
    def _():
        pltpu.sync_copy(acc.at[stripe], b_out.at[stripe])

    @pl.when(cid == 1)
    def _():
        pltpu.sync_copy(acc.at[stripe], s_out.at[stripe])
        pltpu.sync_copy(cntloc, cnt_out.at[wid])


def _run_sc(src_p, dst_p, x, d, c01):
    mesh = plsc.VectorSubcoreMesh(core_axis_name="c", subcore_axis_name="s")
    return pl.kernel(
        _sc_edges,
        out_type=(jax.ShapeDtypeStruct((N_PAD, F), jnp.float32),
                  jax.ShapeDtypeStruct((N_PAD, F), jnp.float32),
                  jax.ShapeDtypeStruct((NS, N_PAD), jnp.float32)),
        mesh=mesh,
        compiler_params=pltpu.CompilerParams(needs_layout_passes=False,
                                             use_tc_tiling_on_sc=False),
        scratch_types=[
            pltpu.VMEM_SHARED((N_PAD, F), jnp.float32),
            pltpu.VMEM((EB, F // 2), jnp.float32),
            pltpu.VMEM((EB, F // 2), jnp.float32),
            pltpu.VMEM((EB, F), jnp.float32),
            pltpu.VMEM((EB,), jnp.int32),
            pltpu.VMEM((EB,), jnp.int32),
            pltpu.VMEM((EB,), jnp.int32),
            pltpu.VMEM((EB,), jnp.int32),
            pltpu.VMEM((EB,), jnp.float32),
            pltpu.VMEM((N_PAD,), jnp.float32),
            pltpu.VMEM((L,), jnp.float32),
            pltpu.SemaphoreType.DMA,
            pltpu.SemaphoreType.DMA,
        ],
    )(src_p, dst_p, x, d, c01)


def kernel(x, edge_index, W, U, c, b):
    W0 = W[:F]
    W1 = W[F:]
    qs = jax.nn.softmax(c)
    wd_t = (W0 - W1).T
    w1_t = W1.T
    ws_t = (qs[0] * W0 + qs[1] * W1).T
    du = U[0] - U[1]
    dum = jnp.zeros((F, 128), jnp.float32).at[:, 0].set(du)
    c01 = jnp.full((L,), c[0] - c[1], jnp.float32)

    src_p = jnp.zeros((E_PAD,), jnp.int32).at[:E].set(edge_index[0])
    dst_p = jnp.zeros((E_PAD,), jnp.int32).at[:E].set(edge_index[1])

    xpk = lax.bitcast_convert_type(
        x.astype(jnp.bfloat16)
        .reshape(N, F // (2 * L), 2, L)
        .transpose(0, 1, 3, 2)
        .reshape(N, F // 2, 2), jnp.float32)

    dmat = pl.pallas_call(
        _d_body,
        grid=(N // 400,),
        in_specs=[pl.BlockSpec((400, F), lambda i: (i, 0)),
                  pl.BlockSpec((F, 128), lambda i: (0, 0))],
        out_specs=pl.BlockSpec((400, 128), lambda i: (i, 0)),
        out_shape=jax.ShapeDtypeStruct((N, 128), jnp.float32),
    )(x, dum)
    d = dmat[:, 0]

    b_acc, s_acc, cnt_parts = _run_sc(src_p, dst_p, xpk, d, c01)

    x_pad = jnp.zeros((N_PAD, F), jnp.float32).at[:N].set(x)

    out = pl.pallas_call(
        _final_body,
        grid=(N_PAD // BLK,),
        in_specs=[pl.BlockSpec((BLK, F), lambda i: (i, 0)),
                  pl.BlockSpec((BLK, F), lambda i: (i, 0)),
                  pl.BlockSpec((BLK, F), lambda i: (i, 0)),
                  pl.BlockSpec((NS, BLK), lambda i: (0, i)),
                  pl.BlockSpec((F, F), lambda i: (0, 0)),
                  pl.BlockSpec((F, F), lambda i: (0, 0)),
                  pl.BlockSpec((F, F), lambda i: (0, 0)),
                  pl.BlockSpec((1, F), lambda i: (0, 0))],
        out_specs=pl.BlockSpec((BLK, F), lambda i: (i, 0)),
        out_shape=jax.ShapeDtypeStruct((N_PAD, F), jnp.float32),
    )(x_pad, b_acc, s_acc, cnt_parts, wd_t, w1_t, ws_t, b.reshape(1, F))
    return out[:N]


# speedup vs baseline: 1.2286x; 1.2286x over previous
import jax
import jax.numpy as jnp
from jax import lax
from jax.experimental import pallas as pl
from jax.experimental.pallas import tpu as pltpu
from jax.experimental.pallas import tpu_sc as plsc

N = 10000
E = 320000
F = 128
NS = 16
L = 16
EB = 128
NB = 158
E_PAD = NB * EB * NS
N_PAD = 10240
RPT = N_PAD // NS
BLK = 512


def _d_body(x_ref, dum_ref, o_ref):
    o_ref[...] = jnp.dot(x_ref[...], dum_ref[...],
                         preferred_element_type=jnp.float32)


def _final_body(x_ref, b_acc_ref, s_acc_ref, cnt_ref, wd_ref, w1_ref, ws_ref,
                bias_ref, o_ref):
    acc = jnp.dot(b_acc_ref[...], wd_ref[...],
                  preferred_element_type=jnp.float32)
    acc = acc + jnp.dot(s_acc_ref[...], w1_ref[...],
                        preferred_element_type=jnp.float32)
    acc = acc + jnp.dot(x_ref[...], ws_ref[...],
                        preferred_element_type=jnp.float32)
    cnt = jnp.sum(cnt_ref[...], axis=0)[:, None]
    conv = acc / (cnt + 1.0) + bias_ref[...]
    o_ref[...] = x_ref[...] + jnp.maximum(conv, 0.0)


def _sc_edges(src_ref, dst_ref, x_ref, d_ref, c01_ref,
              b_out, s_out, cnt_out,
              acc, rows0, rows1, srcb0, srcb1, dstb0, dstb1, wb, dc, c01v,
              gsem0, gsem1):
    cid = lax.axis_index("c")
    wid = lax.axis_index("s")
    rows = (rows0, rows1)
    srcb = (srcb0, srcb1)
    dstb = (dstb0, dstb1)
    gsem = (gsem0, gsem1)
    dloc = dc
    cntloc = dc

    def _zrow(r, _):
        for t in range(F // L):
            rows0[r, pl.ds(t * L, L)] = jnp.zeros((L,), jnp.float32)
        return 0
    lax.fori_loop(0, EB, _zrow, 0)
    for k in range(RPT // EB):
        pltpu.sync_copy(rows0, acc.at[pl.ds(wid * RPT + k * EB, EB)])

    @pl.when(cid == 0)
    def _():
        pltpu.sync_copy(d_ref, dloc.at[pl.ds(0, N)])

    @pl.when(cid == 1)
    def _():
        def _zcnt(r, _):
            cntloc[pl.ds(r * L, L)] = jnp.zeros((L,), jnp.float32)
            return 0
        lax.fori_loop(0, N_PAD // L, _zcnt, 0)

    pltpu.sync_copy(c01_ref, c01v)
    plsc.subcore_barrier()

    base = wid * (NB * EB)

    def _fetch(i, b):
        off = base + i * EB
        pltpu.sync_copy(src_ref.at[pl.ds(off, EB)], srcb[b])
        pltpu.sync_copy(dst_ref.at[pl.ds(off, EB)], dstb[b])
        pltpu.async_copy(x_ref.at[srcb[b]], rows[b], gsem[b])

    def _gwait(b):
        pltpu.make_async_copy(x_ref.at[srcb[b]], rows[b], gsem[b]).wait()

    def _compute_sc0(b):
        c01 = c01v[...]
        for g in range(EB // L):
            sv = srcb[b][pl.ds(g * L, L)]
            dv = dstb[b][pl.ds(g * L, L)]
            dsv = plsc.load_gather(dloc, [sv])
            ddv = plsc.load_gather(dloc, [dv])
            z = dsv - ddv + c01
            w = 1.0 / (1.0 + jnp.exp(-z))
            w = jnp.where(sv == dv, 0.0, w)
            wb[pl.ds(g * L, L)] = w

        def _scale(g, _):
            wv = wb[pl.ds(g * L, L)]
            for j in range(L):
                wj = wv[j]
                row = g * L + j
                for t in range(F // L):
                    sl = pl.ds(t * L, L)
                    rows[b][row, sl] = rows[b][row, sl] * wj
            return 0
        lax.fori_loop(0, EB // L, _scale, 0)

    def _compute_sc1(b):
        ones = jnp.ones((L,), jnp.float32)
        for g in range(EB // L):
            sl = pl.ds(g * L, L)
            sv = srcb[b][sl]
            dv = dstb[b][sl]
            valid = sv != dv
            dstb[b][sl] = jnp.where(valid, dv, N)
            plsc.addupdate_scatter(cntloc, [dv], ones, mask=valid)

    def _make_loop(compute):
        def _body(g, _):
            for b in (0, 1):
                i = 2 * g + b
                _gwait(b)

                @pl.when(i + 1 < NB)
                def _():
                    _fetch(i + 1, 1 - b)
                compute(b)
                pltpu.sync_copy(rows[b], acc.at[dstb[b]], add=True)
            return 0
        return _body

    _fetch(0, 0)

    @pl.when(cid == 0)
    def _():
        lax.fori_loop(0, NB // 2, _make_loop(_compute_sc0), 0)

    @pl.when(cid == 1)
    def _():
        lax.fori_loop(0, NB // 2, _make_loop(_compute_sc1), 0)

    plsc.subcore_barrier()
    stripe = pl.ds(wid * RPT, RPT)

    @pl.when(cid == 0)
    def _():
        pltpu.sync_copy(acc.at[stripe], b_out.at[stripe])

    @pl.when(cid == 1)
    def _():
        pltpu.sync_copy(acc.at[stripe], s_out.at[stripe])
        pltpu.sync_copy(cntloc, cnt_out.at[wid])


def _run_sc(src_p, dst_p, x, d, c01):
    mesh = plsc.VectorSubcoreMesh(core_axis_name="c", subcore_axis_name="s")
    return pl.kernel(
        _sc_edges,
        out_type=(jax.ShapeDtypeStruct((N_PAD, F), jnp.float32),
                  jax.ShapeDtypeStruct((N_PAD, F), jnp.float32),
                  jax.ShapeDtypeStruct((NS, N_PAD), jnp.float32)),
        mesh=mesh,
        compiler_params=pltpu.CompilerParams(needs_layout_passes=False),
        scratch_types=[
            pltpu.VMEM_SHARED((N_PAD, F), jnp.float32),
            pltpu.VMEM((EB, F), jnp.float32),
            pltpu.VMEM((EB, F), jnp.float32),
            pltpu.VMEM((EB,), jnp.int32),
            pltpu.VMEM((EB,), jnp.int32),
            pltpu.VMEM((EB,), jnp.int32),
            pltpu.VMEM((EB,), jnp.int32),
            pltpu.VMEM((EB,), jnp.float32),
            pltpu.VMEM((N_PAD,), jnp.float32),
            pltpu.VMEM((L,), jnp.float32),
            pltpu.SemaphoreType.DMA,
            pltpu.SemaphoreType.DMA,
        ],
    )(src_p, dst_p, x, d, c01)


def kernel(x, edge_index, W, U, c, b):
    W0 = W[:F]
    W1 = W[F:]
    qs = jax.nn.softmax(c)
    wd_t = (W0 - W1).T
    w1_t = W1.T
    ws_t = (qs[0] * W0 + qs[1] * W1).T
    du = U[0] - U[1]
    dum = jnp.zeros((F, 128), jnp.float32).at[:, 0].set(du)
    c01 = jnp.full((L,), c[0] - c[1], jnp.float32)

    src_p = jnp.zeros((E_PAD,), jnp.int32).at[:E].set(edge_index[0])
    dst_p = jnp.zeros((E_PAD,), jnp.int32).at[:E].set(edge_index[1])

    dmat = pl.pallas_call(
        _d_body,
        grid=(N // 400,),
        in_specs=[pl.BlockSpec((400, F), lambda i: (i, 0)),
                  pl.BlockSpec((F, 128), lambda i: (0, 0))],
        out_specs=pl.BlockSpec((400, 128), lambda i: (i, 0)),
        out_shape=jax.ShapeDtypeStruct((N, 128), jnp.float32),
    )(x, dum)
    d = dmat[:, 0]

    b_acc, s_acc, cnt_parts = _run_sc(src_p, dst_p, x, d, c01)

    x_pad = jnp.zeros((N_PAD, F), jnp.float32).at[:N].set(x)

    out = pl.pallas_call(
        _final_body,
        grid=(N_PAD // BLK,),
        in_specs=[pl.BlockSpec((BLK, F), lambda i: (i, 0)),
                  pl.BlockSpec((BLK, F), lambda i: (i, 0)),
                  pl.BlockSpec((BLK, F), lambda i: (i, 0)),
                  pl.BlockSpec((NS, BLK), lambda i: (0, i)),
                  pl.BlockSpec((F, F), lambda i: (0, 0)),
                  pl.BlockSpec((F, F), lambda i: (0, 0)),
                  pl.BlockSpec((F, F), lambda i: (0, 0)),
                  pl.BlockSpec((1, F), lambda i: (0, 0))],
        out_specs=pl.BlockSpec((BLK, F), lambda i: (i, 0)),
        out_shape=jax.ShapeDtypeStruct((N_PAD, F), jnp.float32),
    )(x_pad, b_acc, s_acc, cnt_parts, wd_t, w1_t, ws_t, b.reshape(1, F))
    return out[:N]
